# baseline (speedup 1.0000x reference)
import functools

import jax
import jax.numpy as jnp
from jax import lax
from jax.experimental import pallas as pl
from jax.experimental.pallas import tpu as pltpu
from jax.experimental.pallas import tpu_sc as plsc

NC = 2
NS = 16
NW = NC * NS
CHUNK = 128
IR = 6


def _deg_body(npad, nch, dst1_hbm, degp_hbm, dstc, xtra_v, ones_v, zero_v,
              *rest):
    isems = rest[:IR]
    deg_sh = rest[IR]
    c = lax.axis_index("c")
    s = lax.axis_index("s")
    wid = s * NC + c
    rpt = npad // NS
    base_e = wid * nch * CHUNK

    def idxload(j, u):
        pltpu.async_copy(dst1_hbm.at[pl.ds(base_e + j * CHUNK, CHUNK)],
                         dstc.at[u], isems[u])

    def wait_idx(u):
        pltpu.make_async_copy(dst1_hbm.at[pl.ds(0, CHUNK)], dstc.at[u],
                              isems[u]).wait()

    for i in range(rpt // 16):
        zero_v[pl.ds(i * 16, 16)] = jnp.zeros((16,), jnp.float32)
    pltpu.sync_copy(zero_v, deg_sh.at[pl.ds(s * rpt, rpt)])
    for i in range(CHUNK // 16):
        ones_v[pl.ds(i * 16, 16)] = jnp.ones((16,), jnp.float32)
    for u in range(IR):
        idxload(u, u)
    plsc.subcore_barrier()

    def body(t, _):
        base = t * IR
        for u in range(IR):
            wait_idx(u)
            pltpu.sync_copy(ones_v, deg_sh.at[dstc.at[u]], add=True)
            idxload(base + u + IR, u)
        return _

    lax.fori_loop(0, nch // IR - 1, body, 0)
    for u in range(IR):
        wait_idx(u)
        pltpu.sync_copy(ones_v, deg_sh.at[dstc.at[u]], add=True)

    @pl.when(s == 0)
    def _():
        for e in range(NXTRA):
            off = (NW * nch + NXTRA * c + e) * CHUNK
            pltpu.sync_copy(dst1_hbm.at[pl.ds(off, CHUNK)], xtra_v)
            pltpu.sync_copy(ones_v, deg_sh.at[xtra_v], add=True)

    plsc.subcore_barrier()
    pltpu.sync_copy(deg_sh.at[pl.ds(s * rpt, rpt)],
                    degp_hbm.at[c].at[pl.ds(s * rpt, rpt)])


NXTRA = 2
SCHUNK = 64
SNBUF = 3
SIR = 6
SXTRA = 4


def _scat_body(npad, nch, y_hbm, src1_hbm, dst1_hbm, outp_hbm, srcc, dstc,
               rows_v, *rest):
    gsems = rest[:SNBUF]
    ssems = rest[SNBUF:2 * SNBUF]
    isems = rest[2 * SNBUF:2 * SNBUF + SIR]
    accum_sh = rest[2 * SNBUF + SIR]
    c = lax.axis_index("c")
    s = lax.axis_index("s")
    wid = s * NC + c
    rpt = npad // NS
    base_e = wid * nch * SCHUNK

    pltpu.sync_copy(y_hbm.at[pl.ds(s * rpt, rpt)],
                    accum_sh.at[pl.ds(s * rpt, rpt)])
    plsc.subcore_barrier()

    def idxload(j, u):
        off = base_e + j * SCHUNK
        pltpu.async_copy(src1_hbm.at[pl.ds(off, SCHUNK)], srcc.at[u],
                         isems[u])
        pltpu.async_copy(dst1_hbm.at[pl.ds(off, SCHUNK)], dstc.at[u],
                         isems[u])

    def wait_idx(u):
        pltpu.make_async_copy(src1_hbm.at[pl.ds(0, SCHUNK)], srcc.at[u],
                              isems[u]).wait()
        pltpu.make_async_copy(dst1_hbm.at[pl.ds(0, SCHUNK)], dstc.at[u],
                              isems[u]).wait()

    def gather(u, b):
        pltpu.async_copy(y_hbm.at[srcc.at[u]], rows_v.at[b], gsems[b])

    def wait_gather(b):
        pltpu.make_async_copy(y_hbm.at[srcc.at[0]], rows_v.at[b],
                              gsems[b]).wait()

    def scat(u, b):
        pltpu.async_copy(rows_v.at[b], accum_sh.at[dstc.at[u]], ssems[b],
                         add=True)

    def wait_scat(b):
        pltpu.make_async_copy(rows_v.at[b], accum_sh.at[dstc.at[0]],
                              ssems[b]).wait()

    for u in range(SIR):
        idxload(u, u)
    wait_idx(0)
    gather(0, 0)

    for u in range(SIR):
        wait_gather(u % SNBUF)
        scat(u, u % SNBUF)
        if u >= 2:
            wait_scat((u + 1) % SNBUF)
            idxload(u + 4, (u + 4) % SIR)
        wait_idx((u + 1) % SIR)
        gather((u + 1) % SIR, (u + 1) % SNBUF)

    def body(t, _):
        base = t * SIR
        for u in range(SIR):
            j = base + u
            wait_gather(u % SNBUF)
            scat(u, u % SNBUF)
            wait_scat((u + 1) % SNBUF)
            idxload(j + 4, (u + 4) % SIR)
            wait_idx((u + 1) % SIR)
            gather((u + 1) % SIR, (u + 1) % SNBUF)
        return _

    lax.fori_loop(1, nch // SIR - 1, body, 0)

    for u in range(SIR):
        j = nch - SIR + u
        wait_gather(u % SNBUF)
        scat(u, u % SNBUF)
        wait_scat((u + 1) % SNBUF)
        if u < 2:
            idxload(j + 4, (u + 4) % SIR)
        if u < SIR - 1:
            wait_idx((u + 1) % SIR)
            gather((u + 1) % SIR, (u + 1) % SNBUF)
    wait_scat((nch - 2) % SNBUF)
    wait_scat((nch - 1) % SNBUF)

    @pl.when(s == 0)
    def _():
        for e in range(SXTRA):
            off = (NW * nch + SXTRA * c + e) * SCHUNK
            pltpu.sync_copy(src1_hbm.at[pl.ds(off, SCHUNK)], srcc.at[0])
            pltpu.sync_copy(dst1_hbm.at[pl.ds(off, SCHUNK)], dstc.at[0])
            pltpu.sync_copy(y_hbm.at[srcc.at[0]],
                            rows_v.at[0, pl.ds(0, SCHUNK)])
            pltpu.sync_copy(rows_v.at[0, pl.ds(0, SCHUNK)],
                            accum_sh.at[dstc.at[0]], add=True)

    plsc.subcore_barrier()
    pltpu.sync_copy(accum_sh.at[pl.ds(s * rpt, rpt)],
                    outp_hbm.at[c].at[pl.ds(s * rpt, rpt)])


def _split_body(ei_ref, src_ref, dst_ref):
    src_ref[...] = ei_ref[0]
    dst_ref[...] = ei_ref[1]


def _xw_body(x_ref, w_ref, xw_ref):
    xw_ref[...] = jnp.dot(x_ref[...], w_ref[...],
                          preferred_element_type=jnp.float32)


def _scale_body(xw_ref, degp_ref, y_ref, dinv_ref):
    deg = degp_ref[0] + degp_ref[1] + 1.0
    dinv = lax.rsqrt(deg)[:, None]
    y_ref[...] = xw_ref[...] * dinv
    dinv_ref[...] = dinv


def _fin_body(outp_ref, y_ref, dinv_ref, b_ref, out_ref):
    acc = outp_ref[0] + outp_ref[1] - y_ref[...]
    out_ref[...] = jnp.maximum(acc * dinv_ref[...] + b_ref[...], 0.0)


def kernel(x, edge_index, W, b):
    N, D = x.shape
    E = edge_index.shape[1]
    NPAD = ((N + NS * 16 - 1) // (NS * 16)) * (NS * 16)
    NPAD = max(NPAD, ((N + 127) // 128) * 128)
    NROWS = E // CHUNK
    nch = (NROWS // NW) // IR * IR
    assert NROWS - NW * nch == NC * NXTRA
    SROWS = E // SCHUNK
    nchs = (SROWS // NW) // SIR * SIR
    assert SROWS - NW * nchs == NC * SXTRA
    assert nchs // SIR >= 3

    ei32 = edge_index.astype(jnp.int32)
    EB = 32768
    src1, dst1 = pl.pallas_call(
        _split_body,
        grid=(-(-E // EB),),
        in_specs=[pl.BlockSpec((2, EB), lambda i: (0, i))],
        out_specs=[
            pl.BlockSpec((EB,), lambda i: (i,)),
            pl.BlockSpec((EB,), lambda i: (i,)),
        ],
        out_shape=[
            jax.ShapeDtypeStruct((E,), jnp.int32),
            jax.ShapeDtypeStruct((E,), jnp.int32),
        ],
    )(ei32)

    mesh = plsc.VectorSubcoreMesh(core_axis_name="c", subcore_axis_name="s",
                                  num_cores=NC, num_subcores=NS)

    degp = pl.kernel(
        functools.partial(_deg_body, NPAD, nch),
        out_type=jax.ShapeDtypeStruct((NC, NPAD), jnp.float32),
        mesh=mesh,
        scratch_types=[
            pltpu.VMEM((IR, CHUNK), jnp.int32),
            pltpu.VMEM((CHUNK,), jnp.int32),
            pltpu.VMEM((CHUNK,), jnp.float32),
            pltpu.VMEM((NPAD // NS,), jnp.float32),
        ] + [pltpu.SemaphoreType.DMA] * IR + [
            pltpu.VMEM_SHARED((NPAD,), jnp.float32),
        ],
    )(dst1)

    BR = NPAD // 8
    xw = pl.pallas_call(
        _xw_body,
        grid=(8,),
        in_specs=[
            pl.BlockSpec((BR, D), lambda i: (i, 0)),
            pl.BlockSpec((D, D), lambda i: (0, 0)),
        ],
        out_specs=pl.BlockSpec((BR, D), lambda i: (i, 0)),
        out_shape=jax.ShapeDtypeStruct((NPAD, D), jnp.float32),
    )(x, W)

    y_pad, dinv = pl.pallas_call(
        _scale_body,
        grid=(8,),
        in_specs=[
            pl.BlockSpec((BR, D), lambda i: (i, 0)),
            pl.BlockSpec((NC, BR), lambda i: (0, i)),
        ],
        out_specs=[
            pl.BlockSpec((BR, D), lambda i: (i, 0)),
            pl.BlockSpec((BR, 1), lambda i: (i, 0)),
        ],
        out_shape=[
            jax.ShapeDtypeStruct((NPAD, D), jnp.float32),
            jax.ShapeDtypeStruct((NPAD, 1), jnp.float32),
        ],
    )(xw, degp)

    outp = pl.kernel(
        functools.partial(_scat_body, NPAD, nchs),
        out_type=jax.ShapeDtypeStruct((NC, NPAD, D), jnp.float32),
        mesh=mesh,
        scratch_types=[
            pltpu.VMEM((SIR, SCHUNK), jnp.int32),
            pltpu.VMEM((SIR, SCHUNK), jnp.int32),
            pltpu.VMEM((SNBUF, SCHUNK, D), jnp.float32),
        ] + [pltpu.SemaphoreType.DMA] * (2 * SNBUF + SIR) + [
            pltpu.VMEM_SHARED((NPAD, D), jnp.float32),
        ],
    )(y_pad, src1, dst1)

    RB = 2000
    out = pl.pallas_call(
        _fin_body,
        grid=(N // RB,),
        in_specs=[
            pl.BlockSpec((NC, RB, D), lambda i: (0, i, 0)),
            pl.BlockSpec((RB, D), lambda i: (i, 0)),
            pl.BlockSpec((RB, 1), lambda i: (i, 0)),
            pl.BlockSpec((1, D), lambda i: (0, 0)),
        ],
        out_specs=pl.BlockSpec((RB, D), lambda i: (i, 0)),
        out_shape=jax.ShapeDtypeStruct((N, D), jnp.float32),
    )(outp, y_pad, dinv, b.reshape(1, D))
    return out


# speedup vs baseline: 1.4408x; 1.4408x over previous
import functools

import jax
import jax.numpy as jnp
from jax import lax
from jax.experimental import pallas as pl
from jax.experimental.pallas import tpu as pltpu
from jax.experimental.pallas import tpu_sc as plsc

NC = 2
NS = 16
NW = NC * NS
CHUNK = 128
IR = 6


def _deg_body(npad, nch, dst1_hbm, degp_hbm, dstc, xtra_v, ones_v, zero_v,
              *rest):
    isems = rest[:IR]
    deg_sh = rest[IR]
    c = lax.axis_index("c")
    s = lax.axis_index("s")
    wid = s * NC + c
    rpt = npad // NS
    base_e = wid * nch * CHUNK

    def idxload(j, u):
        pltpu.async_copy(dst1_hbm.at[pl.ds(base_e + j * CHUNK, CHUNK)],
                         dstc.at[u], isems[u])

    def wait_idx(u):
        pltpu.make_async_copy(dst1_hbm.at[pl.ds(0, CHUNK)], dstc.at[u],
                              isems[u]).wait()

    for i in range(rpt // 16):
        zero_v[pl.ds(i * 16, 16)] = jnp.zeros((16,), jnp.float32)
    pltpu.sync_copy(zero_v, deg_sh.at[pl.ds(s * rpt, rpt)])
    for i in range(CHUNK // 16):
        ones_v[pl.ds(i * 16, 16)] = jnp.ones((16,), jnp.float32)
    for u in range(IR):
        idxload(u, u)
    plsc.subcore_barrier()

    def body(t, _):
        base = t * IR
        for u in range(IR):
            wait_idx(u)
            pltpu.sync_copy(ones_v, deg_sh.at[dstc.at[u]], add=True)
            idxload(base + u + IR, u)
        return _

    lax.fori_loop(0, nch // IR - 1, body, 0)
    for u in range(IR):
        wait_idx(u)
        pltpu.sync_copy(ones_v, deg_sh.at[dstc.at[u]], add=True)

    @pl.when(s == 0)
    def _():
        for e in range(NXTRA):
            off = (NW * nch + NXTRA * c + e) * CHUNK
            pltpu.sync_copy(dst1_hbm.at[pl.ds(off, CHUNK)], xtra_v)
            pltpu.sync_copy(ones_v, deg_sh.at[xtra_v], add=True)

    plsc.subcore_barrier()
    pltpu.sync_copy(deg_sh.at[pl.ds(s * rpt, rpt)],
                    degp_hbm.at[c].at[pl.ds(s * rpt, rpt)])


NXTRA = 2
SCHUNK = 128
SNBUF = 2
SIR = 6
SXTRA = 2


def _scat_body(npad, nch, y_hbm, src1_hbm, dst1_hbm, outp_hbm, srcc, dstc,
               rows_v, *rest):
    gsems = rest[:SNBUF]
    ssems = rest[SNBUF:2 * SNBUF]
    isems = rest[2 * SNBUF:2 * SNBUF + SIR]
    accum_sh = rest[2 * SNBUF + SIR]
    c = lax.axis_index("c")
    s = lax.axis_index("s")
    wid = s * NC + c
    rpt = npad // NS
    base_e = wid * nch * SCHUNK

    pltpu.sync_copy(y_hbm.at[pl.ds(s * rpt, rpt)],
                    accum_sh.at[pl.ds(s * rpt, rpt)])
    plsc.subcore_barrier()

    def idxload(j, u):
        off = base_e + j * SCHUNK
        pltpu.async_copy(src1_hbm.at[pl.ds(off, SCHUNK)], srcc.at[u],
                         isems[u])
        pltpu.async_copy(dst1_hbm.at[pl.ds(off, SCHUNK)], dstc.at[u],
                         isems[u])

    def wait_idx(u):
        pltpu.make_async_copy(src1_hbm.at[pl.ds(0, SCHUNK)], srcc.at[u],
                              isems[u]).wait()
        pltpu.make_async_copy(dst1_hbm.at[pl.ds(0, SCHUNK)], dstc.at[u],
                              isems[u]).wait()

    def gather(u, b):
        pltpu.async_copy(y_hbm.at[srcc.at[u]], rows_v.at[b], gsems[b])

    def wait_gather(b):
        pltpu.make_async_copy(y_hbm.at[srcc.at[0]], rows_v.at[b],
                              gsems[b]).wait()

    def scat(u, b):
        pltpu.async_copy(rows_v.at[b], accum_sh.at[dstc.at[u]], ssems[b],
                         add=True)

    def wait_scat(b):
        pltpu.make_async_copy(rows_v.at[b], accum_sh.at[dstc.at[0]],
                              ssems[b]).wait()

    for u in range(SIR):
        idxload(u, u)
    for j in range(SNBUF):
        wait_idx(j)
        gather(j, j)

    def body(t, _):
        base = t * SIR
        for u in range(SIR):
            j = base + u
            b = u % SNBUF
            wait_gather(b)
            scat(u, b)
            wait_scat(b)
            idxload(j + SIR, u)
            wait_idx((u + SNBUF) % SIR)
            gather((u + SNBUF) % SIR, b)
        return _

    lax.fori_loop(0, nch // SIR - 1, body, 0)

    for u in range(SIR):
        b = u % SNBUF
        wait_gather(b)
        scat(u, b)
        wait_scat(b)
        if u + SNBUF < SIR:
            wait_idx(u + SNBUF)
            gather(u + SNBUF, b)

    @pl.when(s == 0)
    def _():
        for e in range(SXTRA):
            off = (NW * nch + SXTRA * c + e) * SCHUNK
            pltpu.sync_copy(src1_hbm.at[pl.ds(off, SCHUNK)], srcc.at[0])
            pltpu.sync_copy(dst1_hbm.at[pl.ds(off, SCHUNK)], dstc.at[0])
            pltpu.sync_copy(y_hbm.at[srcc.at[0]],
                            rows_v.at[0, pl.ds(0, SCHUNK)])
            pltpu.sync_copy(rows_v.at[0, pl.ds(0, SCHUNK)],
                            accum_sh.at[dstc.at[0]], add=True)

    plsc.subcore_barrier()
    pltpu.sync_copy(accum_sh.at[pl.ds(s * rpt, rpt)],
                    outp_hbm.at[c].at[pl.ds(s * rpt, rpt)])


def _split_body(ei_ref, src_ref, dst_ref):
    src_ref[...] = ei_ref[0]
    dst_ref[...] = ei_ref[1]


def _xw_body(x_ref, w_ref, xw_ref):
    xw_ref[...] = jnp.dot(x_ref[...], w_ref[...],
                          preferred_element_type=jnp.float32)


def _scale_body(xw_ref, degp_ref, y_ref, dinv_ref):
    deg = degp_ref[0] + degp_ref[1] + 1.0
    dinv = lax.rsqrt(deg)[:, None]
    y_ref[...] = xw_ref[...] * dinv
    dinv_ref[...] = dinv


def _fin_body(outp_ref, y_ref, dinv_ref, b_ref, out_ref):
    acc = outp_ref[0] + outp_ref[1] - y_ref[...]
    out_ref[...] = jnp.maximum(acc * dinv_ref[...] + b_ref[...], 0.0)


def kernel(x, edge_index, W, b):
    N, D = x.shape
    E = edge_index.shape[1]
    NPAD = ((N + NS * 16 - 1) // (NS * 16)) * (NS * 16)
    NPAD = max(NPAD, ((N + 127) // 128) * 128)
    NROWS = E // CHUNK
    nch = (NROWS // NW) // IR * IR
    assert NROWS - NW * nch == NC * NXTRA
    SROWS = E // SCHUNK
    nchs = (SROWS // NW) // SIR * SIR
    assert SROWS - NW * nchs == NC * SXTRA
    assert nchs // SIR >= 3

    ei32 = edge_index.astype(jnp.int32)
    EB = 32768
    src1, dst1 = pl.pallas_call(
        _split_body,
        grid=(-(-E // EB),),
        in_specs=[pl.BlockSpec((2, EB), lambda i: (0, i))],
        out_specs=[
            pl.BlockSpec((EB,), lambda i: (i,)),
            pl.BlockSpec((EB,), lambda i: (i,)),
        ],
        out_shape=[
            jax.ShapeDtypeStruct((E,), jnp.int32),
            jax.ShapeDtypeStruct((E,), jnp.int32),
        ],
    )(ei32)

    mesh = plsc.VectorSubcoreMesh(core_axis_name="c", subcore_axis_name="s",
                                  num_cores=NC, num_subcores=NS)

    degp = pl.kernel(
        functools.partial(_deg_body, NPAD, nch),
        out_type=jax.ShapeDtypeStruct((NC, NPAD), jnp.float32),
        mesh=mesh,
        scratch_types=[
            pltpu.VMEM((IR, CHUNK), jnp.int32),
            pltpu.VMEM((CHUNK,), jnp.int32),
            pltpu.VMEM((CHUNK,), jnp.float32),
            pltpu.VMEM((NPAD // NS,), jnp.float32),
        ] + [pltpu.SemaphoreType.DMA] * IR + [
            pltpu.VMEM_SHARED((NPAD,), jnp.float32),
        ],
    )(dst1)

    BR = NPAD // 8
    xw = pl.pallas_call(
        _xw_body,
        grid=(8,),
        in_specs=[
            pl.BlockSpec((BR, D), lambda i: (i, 0)),
            pl.BlockSpec((D, D), lambda i: (0, 0)),
        ],
        out_specs=pl.BlockSpec((BR, D), lambda i: (i, 0)),
        out_shape=jax.ShapeDtypeStruct((NPAD, D), jnp.float32),
    )(x, W)

    y_pad, dinv = pl.pallas_call(
        _scale_body,
        grid=(8,),
        in_specs=[
            pl.BlockSpec((BR, D), lambda i: (i, 0)),
            pl.BlockSpec((NC, BR), lambda i: (0, i)),
        ],
        out_specs=[
            pl.BlockSpec((BR, D), lambda i: (i, 0)),
            pl.BlockSpec((BR, 1), lambda i: (i, 0)),
        ],
        out_shape=[
            jax.ShapeDtypeStruct((NPAD, D), jnp.float32),
            jax.ShapeDtypeStruct((NPAD, 1), jnp.float32),
        ],
    )(xw, degp)

    outp = pl.kernel(
        functools.partial(_scat_body, NPAD, nchs),
        out_type=jax.ShapeDtypeStruct((NC, NPAD, D), jnp.float32),
        mesh=mesh,
        scratch_types=[
            pltpu.VMEM((SIR, SCHUNK), jnp.int32),
            pltpu.VMEM((SIR, SCHUNK), jnp.int32),
            pltpu.VMEM((SNBUF, SCHUNK, D), jnp.float32),
        ] + [pltpu.SemaphoreType.DMA] * (2 * SNBUF + SIR) + [
            pltpu.VMEM_SHARED((NPAD, D), jnp.float32),
        ],
    )(y_pad, src1, dst1)

    RB = 2000
    out = pl.pallas_call(
        _fin_body,
        grid=(N // RB,),
        in_specs=[
            pl.BlockSpec((NC, RB, D), lambda i: (0, i, 0)),
            pl.BlockSpec((RB, D), lambda i: (i, 0)),
            pl.BlockSpec((RB, 1), lambda i: (i, 0)),
            pl.BlockSpec((1, D), lambda i: (0, 0)),
        ],
        out_specs=pl.BlockSpec((RB, D), lambda i: (i, 0)),
        out_shape=jax.ShapeDtypeStruct((N, D), jnp.float32),
    )(outp, y_pad, dinv, b.reshape(1, D))
    return out
